# baseline (speedup 1.0000x reference)
import functools

import jax
import jax.numpy as jnp
from jax import lax
from jax.experimental import pallas as pl
from jax.experimental.pallas import tpu as pltpu
from jax.experimental.pallas import tpu_sc as plsc

B = 16384
D = 64
L = 16
NC = 2
NS = 16
NW = NC * NS
BPW = B // NW

_mesh = plsc.VectorSubcoreMesh(core_axis_name="c", subcore_axis_name="s")


@functools.partial(
    pl.kernel,
    mesh=_mesh,
    out_type=jax.ShapeDtypeStruct((B,), jnp.float32),
    scratch_types=[
        pltpu.VMEM((BPW,), jnp.int32),
        pltpu.VMEM((BPW,), jnp.int32),
        pltpu.VMEM((BPW, D), jnp.float32),
        pltpu.VMEM((BPW, D), jnp.float32),
        pltpu.VMEM((L, L), jnp.float32),
        pltpu.VMEM((BPW,), jnp.float32),
        pltpu.VMEM((L,), jnp.float32),
        pltpu.SemaphoreType.DMA,
        pltpu.SemaphoreType.DMA,
    ],
    compiler_params=pltpu.CompilerParams(needs_layout_passes=False,
                                         use_tc_tiling_on_sc=False),
)
def _mf_forward(uidx_hbm, nidx_hbm, uf_hbm, nf_hbm, gi_hbm, out_hbm,
                uidx_v, nidx_v, urows, nrows, pscr, out_v, gi_v,
                sem_u, sem_n):
    wid = lax.axis_index("s") * NC + lax.axis_index("c")
    base = wid * BPW

    pltpu.sync_copy(uidx_hbm.at[pl.ds(base, BPW)], uidx_v)
    pltpu.sync_copy(nidx_hbm.at[pl.ds(base, BPW)], nidx_v)
    pltpu.sync_copy(gi_hbm, gi_v)

    cp_u = pltpu.async_copy(uf_hbm.at[uidx_v], urows, sem_u)
    cp_n = pltpu.async_copy(nf_hbm.at[nidx_v], nrows, sem_n)
    cp_u.wait()
    cp_n.wait()

    gvec = gi_v[...]
    lane = lax.iota(jnp.int32, L)

    def group(g, _):
        for i in range(L):
            r = g * L + i
            acc = urows[r, pl.ds(0, L)] * nrows[r, pl.ds(0, L)]
            for k in range(1, D // L):
                acc = acc + urows[r, pl.ds(k * L, L)] * nrows[r, pl.ds(k * L, L)]
            pscr[i, :] = acc
        o = gvec
        for c in range(L):
            o = o + plsc.load_gather(pscr, [lane, jnp.full((L,), c, jnp.int32)])
        out_v[pl.ds(g * L, L)] = o
        return 0

    lax.fori_loop(0, BPW // L, group, 0)
    pltpu.sync_copy(out_v, out_hbm.at[pl.ds(base, BPW)])


def kernel(user_idxs, note_idxs, user_factors, note_factors, global_intercept):
    gi16 = jnp.broadcast_to(jnp.reshape(global_intercept, (1,)), (L,))
    return _mf_forward(user_idxs.astype(jnp.int32), note_idxs.astype(jnp.int32),
                       user_factors, note_factors, gi16)
